# baseline (speedup 1.0000x reference)
import functools

import jax
import jax.numpy as jnp
from jax import lax
from jax.experimental import pallas as pl
from jax.experimental.pallas import tpu as pltpu
from jax.experimental.pallas import tpu_sc as plsc

D = 64
EPS = 1e-5

_info = plsc.get_sparse_core_info()
NC, NS = _info.num_cores, _info.num_subcores
NW = NC * NS

CHUNK = 256
IDXW = 128
GPC = CHUNK // IDXW
STAGE = 1024
CPS = STAGE // CHUNK
SROWS = STAGE // IDXW

BB = 256


def _make_gather(n_rows):
    assert n_rows % (NW * STAGE) == 0
    rows_per_w = n_rows // NW
    n_groups = rows_per_w // STAGE
    mesh = plsc.VectorSubcoreMesh(core_axis_name="c", subcore_axis_name="s")

    @functools.partial(
        pl.kernel,
        mesh=mesh,
        compiler_params=pltpu.CompilerParams(use_tc_tiling_on_sc=False),
        out_type=jax.ShapeDtypeStruct((n_rows, D), jnp.float32),
        scratch_types=[
            pltpu.VMEM((SROWS, IDXW), jnp.int32),
            pltpu.VMEM((CHUNK, D), jnp.float32),
            pltpu.SemaphoreType.DMA,
        ],
    )
    def gather_k(x_hbm, w_hbm, out_hbm, idx_v, rows_v, sem):
        wid = lax.axis_index("s") * NC + lax.axis_index("c")
        base = wid * rows_per_w

        def group_body(g, _):
            grow0 = base + g * STAGE
            goff = pl.multiple_of(grow0 // IDXW, 8)
            pltpu.sync_copy(x_hbm.at[pl.ds(goff, SROWS)], idx_v)
            for c in range(CPS):
                row0 = grow0 + c * CHUNK
                for j in range(GPC):
                    pltpu.async_copy(
                        w_hbm.at[idx_v.at[c * GPC + j]],
                        rows_v.at[pl.ds(j * IDXW, IDXW)], sem).wait()
                pltpu.sync_copy(rows_v, out_hbm.at[pl.ds(row0, CHUNK)])
            return 0

        lax.fori_loop(0, n_groups, group_body, 0)

    return gather_k


def _ln_tc(mid_ref, lnw_ref, lnb_ref, out_ref):
    x = mid_ref[...]
    left = x[:, :D]
    right = x[:, D:]

    def norm(v):
        mean = jnp.mean(v, axis=-1, keepdims=True)
        var = jnp.mean(jnp.square(v - mean), axis=-1, keepdims=True)
        return (v - mean) * lax.rsqrt(var + EPS)

    nl = norm(left)[:, None, :]
    nr = norm(right)[:, None, :]
    pair = jnp.concatenate([nl, nr], axis=1)
    out = pair.reshape(BB, 50, D)
    out_ref[...] = out * lnw_ref[...] + lnb_ref[...]


def _make_ln(n_rows, h):
    n_lines = n_rows // 2
    lines_pb = n_lines // (n_rows // h)
    nb = n_rows // h
    grid = nb // BB

    return pl.pallas_call(
        _ln_tc,
        grid=(grid,),
        in_specs=[
            pl.BlockSpec((BB * lines_pb, 128), lambda i: (i, 0)),
            pl.BlockSpec((D,), lambda i: (0,)),
            pl.BlockSpec((D,), lambda i: (0,)),
        ],
        out_specs=pl.BlockSpec((BB, h, D), lambda i: (i, 0, 0)),
        out_shape=jax.ShapeDtypeStruct((nb, h, D), jnp.float32),
    )


def kernel(x, weight, ln_weight, ln_bias):
    b, h = x.shape
    n = b * h
    x2 = x.reshape(n // IDXW, IDXW).astype(jnp.int32)
    mid = _make_gather(n)(x2, weight)
    mid2 = mid.reshape(n // 2, 128)
    return _make_ln(n, h)(mid2, ln_weight, ln_bias)
